# Initial kernel scaffold; baseline (speedup 1.0000x reference)
import jax
import jax.numpy as jnp
from jax.experimental import pallas as pl


def kernel(edge_index, h, W_self, b_self, W_neigh, b_neigh):
    raise NotImplementedError("write your pallas kernel here")


# speedup vs baseline: 3.0460x; 3.0460x over previous
import functools

import jax
import jax.numpy as jnp
from jax import lax
from jax.experimental import pallas as pl
from jax.experimental.pallas import tpu as pltpu
from jax.experimental.pallas import tpu_sc as plsc

N = 10000
E = 320000
D = 128

NC = 2
NS = 16
NW = NC * NS

CHUNK = 128
CPW = 80
E_PAD = NW * CPW * CHUNK
ACC_ROWS = 10240
ZROWS = ACC_ROWS // NS
OCHUNK = 80


def _sc_body(src_hbm, dst_hbm, h_hbm, partial_hbm,
             src_v, dst_v, rows_v, acc_sh, sem):
    c = lax.axis_index("c")
    s = lax.axis_index("s")
    wid = s * NC + c

    z16 = jnp.zeros((16,), jnp.float32)

    def zero_body(i, carry):
        r = i // (D // 16)
        col = (i % (D // 16)) * 16
        rows_v[r, pl.ds(col, 16)] = z16
        return carry

    lax.fori_loop(0, CHUNK * (D // 16), zero_body, 0)
    for k in range(ZROWS // CHUNK):
        pltpu.sync_copy(rows_v, acc_sh.at[pl.ds(s * ZROWS + k * CHUNK, CHUNK)])
    plsc.subcore_barrier()

    pltpu.sync_copy(src_hbm.at[pl.ds(wid * CPW, CPW)], src_v)
    pltpu.sync_copy(dst_hbm.at[pl.ds(wid * CPW, CPW)], dst_v)

    def chunk_body(j, carry):
        pltpu.async_copy(h_hbm.at[src_v.at[j]], rows_v, sem).wait()
        pltpu.sync_copy(rows_v, acc_sh.at[dst_v.at[j]], add=True)
        return carry

    lax.fori_loop(0, CPW, chunk_body, 0)
    plsc.subcore_barrier()

    def out_body(i, carry):
        off = (s + i * NS) * OCHUNK
        pltpu.sync_copy(acc_sh.at[pl.ds(off, OCHUNK)], rows_v.at[pl.ds(0, OCHUNK)])
        pltpu.sync_copy(rows_v.at[pl.ds(0, OCHUNK)], partial_hbm.at[c, pl.ds(off, OCHUNK)])
        return carry

    nunits = (N // OCHUNK - s + NS - 1) // NS
    lax.fori_loop(0, nunits, out_body, 0)


def _sc_segment_sum(src2d, dst2d, h):
    mesh = plsc.VectorSubcoreMesh(core_axis_name="c", subcore_axis_name="s")
    kern = pl.kernel(
        _sc_body,
        mesh=mesh,
        out_type=jax.ShapeDtypeStruct((NC, N, D), jnp.float32),
        scratch_types=[
            pltpu.VMEM((CPW, CHUNK), jnp.int32),
            pltpu.VMEM((CPW, CHUNK), jnp.int32),
            pltpu.VMEM((CHUNK, D), jnp.float32),
            pltpu.VMEM_SHARED((ACC_ROWS, D), jnp.float32),
            pltpu.SemaphoreType.DMA,
        ],
    )
    return kern(src2d, dst2d, h)


def _tc_body(h_ref, p0_ref, p1_ref, ws_ref, wn_ref, b_ref, o_ref):
    dn = (((1,), (1,)), ((), ()))
    o_ref[...] = (
        lax.dot_general(h_ref[...], ws_ref[...], dn,
                        preferred_element_type=jnp.float32)
        + lax.dot_general(p0_ref[...] + p1_ref[...], wn_ref[...], dn,
                          preferred_element_type=jnp.float32)
        + b_ref[...]
    )


def _tc_combine(h, p0, p1, W_self, W_neigh, bsum):
    BR = 1000
    return pl.pallas_call(
        _tc_body,
        grid=(N // BR,),
        in_specs=[
            pl.BlockSpec((BR, D), lambda i: (i, 0)),
            pl.BlockSpec((BR, D), lambda i: (i, 0)),
            pl.BlockSpec((BR, D), lambda i: (i, 0)),
            pl.BlockSpec((D, D), lambda i: (0, 0)),
            pl.BlockSpec((D, D), lambda i: (0, 0)),
            pl.BlockSpec((1, D), lambda i: (0, 0)),
        ],
        out_specs=pl.BlockSpec((BR, D), lambda i: (i, 0)),
        out_shape=jax.ShapeDtypeStruct((N, D), jnp.float32),
    )(h, p0, p1, W_self, W_neigh, bsum)


def kernel(edge_index, h, W_self, b_self, W_neigh, b_neigh):
    pad = E_PAD - E
    src = jnp.concatenate([edge_index[0], jnp.zeros((pad,), jnp.int32)])
    pad_dst = N + (jnp.arange(pad, dtype=jnp.int32) % (ACC_ROWS - N))
    dst = jnp.concatenate([edge_index[1], pad_dst])
    src2d = src.reshape(E_PAD // CHUNK, CHUNK)
    dst2d = dst.reshape(E_PAD // CHUNK, CHUNK)
    partial = _sc_segment_sum(src2d, dst2d, h)
    bsum = (b_self + b_neigh).reshape(1, D)
    return _tc_combine(h, partial[0], partial[1], W_self, W_neigh, bsum)


# speedup vs baseline: 3.2459x; 1.0656x over previous
import functools

import jax
import jax.numpy as jnp
from jax import lax
from jax.experimental import pallas as pl
from jax.experimental.pallas import tpu as pltpu
from jax.experimental.pallas import tpu_sc as plsc

N = 10000
E = 320000
D = 128

NC = 2
NS = 16
NW = NC * NS

CHUNK = 128
CPW = 80
E_PAD = NW * CPW * CHUNK
ACC_ROWS = 10240
ZROWS = ACC_ROWS // NS
OCHUNK = 80


def _sc_body(src_hbm, dst_hbm, h_hbm, partial_hbm,
             src_v, dst_v, rows_v, rows_b, acc_sh, sem, sem_b):
    c = lax.axis_index("c")
    s = lax.axis_index("s")
    wid = s * NC + c

    z16 = jnp.zeros((16,), jnp.float32)

    def zero_body(i, carry):
        r = i // (D // 16)
        col = (i % (D // 16)) * 16
        rows_v[r, pl.ds(col, 16)] = z16
        return carry

    lax.fori_loop(0, CHUNK * (D // 16), zero_body, 0)
    for k in range(ZROWS // CHUNK):
        pltpu.sync_copy(rows_v, acc_sh.at[pl.ds(s * ZROWS + k * CHUNK, CHUNK)])
    plsc.subcore_barrier()

    HALF = CPW // 2
    for half in range(2):
        base = wid * CPW + half * HALF
        pltpu.sync_copy(src_hbm.at[pl.ds(base, HALF)], src_v)
        pltpu.sync_copy(dst_hbm.at[pl.ds(base, HALF)], dst_v)
        bufs = (rows_v, rows_b)
        sems = (sem, sem_b)
        desc = [None] * HALF
        desc[0] = pltpu.async_copy(h_hbm.at[src_v.at[0]], bufs[0], sems[0])
        for j in range(HALF):
            buf = bufs[j % 2]
            desc[j].wait()
            if j + 1 < HALF:
                desc[j + 1] = pltpu.async_copy(
                    h_hbm.at[src_v.at[j + 1]], bufs[(j + 1) % 2], sems[(j + 1) % 2])
            pltpu.sync_copy(buf, acc_sh.at[dst_v.at[j]], add=True)
    plsc.subcore_barrier()

    def out_body(i, carry):
        off = (s + i * NS) * OCHUNK
        pltpu.sync_copy(acc_sh.at[pl.ds(off, OCHUNK)], rows_v.at[pl.ds(0, OCHUNK)])
        pltpu.sync_copy(rows_v.at[pl.ds(0, OCHUNK)], partial_hbm.at[c, pl.ds(off, OCHUNK)])
        return carry

    nunits = (N // OCHUNK - s + NS - 1) // NS
    lax.fori_loop(0, nunits, out_body, 0)


def _sc_segment_sum(src2d, dst2d, h):
    mesh = plsc.VectorSubcoreMesh(core_axis_name="c", subcore_axis_name="s")
    kern = pl.kernel(
        _sc_body,
        mesh=mesh,
        out_type=jax.ShapeDtypeStruct((NC, N, D), jnp.float32),
        scratch_types=[
            pltpu.VMEM((CPW // 2, CHUNK), jnp.int32),
            pltpu.VMEM((CPW // 2, CHUNK), jnp.int32),
            pltpu.VMEM((CHUNK, D), jnp.float32),
            pltpu.VMEM((CHUNK, D), jnp.float32),
            pltpu.VMEM_SHARED((ACC_ROWS, D), jnp.float32),
            pltpu.SemaphoreType.DMA,
            pltpu.SemaphoreType.DMA,
        ],
    )
    return kern(src2d, dst2d, h)


def _tc_body(h_ref, p0_ref, p1_ref, ws_ref, wn_ref, b_ref, o_ref):
    dn = (((1,), (1,)), ((), ()))
    o_ref[...] = (
        lax.dot_general(h_ref[...], ws_ref[...], dn,
                        preferred_element_type=jnp.float32)
        + lax.dot_general(p0_ref[...] + p1_ref[...], wn_ref[...], dn,
                          preferred_element_type=jnp.float32)
        + b_ref[...]
    )


def _tc_combine(h, p0, p1, W_self, W_neigh, bsum):
    BR = 1000
    return pl.pallas_call(
        _tc_body,
        grid=(N // BR,),
        in_specs=[
            pl.BlockSpec((BR, D), lambda i: (i, 0)),
            pl.BlockSpec((BR, D), lambda i: (i, 0)),
            pl.BlockSpec((BR, D), lambda i: (i, 0)),
            pl.BlockSpec((D, D), lambda i: (0, 0)),
            pl.BlockSpec((D, D), lambda i: (0, 0)),
            pl.BlockSpec((1, D), lambda i: (0, 0)),
        ],
        out_specs=pl.BlockSpec((BR, D), lambda i: (i, 0)),
        out_shape=jax.ShapeDtypeStruct((N, D), jnp.float32),
    )(h, p0, p1, W_self, W_neigh, bsum)


def kernel(edge_index, h, W_self, b_self, W_neigh, b_neigh):
    pad = E_PAD - E
    src = jnp.concatenate([edge_index[0], jnp.zeros((pad,), jnp.int32)])
    pad_dst = N + (jnp.arange(pad, dtype=jnp.int32) % (ACC_ROWS - N))
    dst = jnp.concatenate([edge_index[1], pad_dst])
    src2d = src.reshape(E_PAD // CHUNK, CHUNK)
    dst2d = dst.reshape(E_PAD // CHUNK, CHUNK)
    partial = _sc_segment_sum(src2d, dst2d, h)
    bsum = (b_self + b_neigh).reshape(1, D)
    return _tc_combine(h, partial[0], partial[1], W_self, W_neigh, bsum)
